# baseline (speedup 1.0000x reference)
import functools

import jax
import jax.numpy as jnp
from jax import lax
from jax.experimental import pallas as pl
from jax.experimental.pallas import tpu as pltpu
from jax.experimental.pallas import tpu_sc as plsc

N = 10000
E = 320000
HID = 128
DE = 16
G = 16

NP = 10240
NC = 2
NS = 16
NW = NC * NS
ET = E // NW
CH = 80
NCH = ET // CH
RPT = NP // NS

EP = 327680
ETP = EP // NW
H = 32
NB = ETP // (2 * H)

_f32 = jnp.float32
_i32 = jnp.int32
_HI = jax.lax.Precision.HIGHEST


def _sc_mesh():
    return plsc.VectorSubcoreMesh(core_axis_name="c", subcore_axis_name="s")


def _p0_body(src_h, dst_h, ea_h, cntp_h, asump_h,
             srcv, dstv, dmod, eav16, eav, cnt_loc, zb, asum_sh, sem0):
    cid = lax.axis_index("c")
    sid = lax.axis_index("s")
    wid = cid * NS + sid
    row0 = sid * RPT
    zero16 = jnp.zeros((16,), _f32)
    ones16 = zero16 + 1.0

    def _z1(i, _):
        cnt_loc[pl.ds(i * 16, 16)] = zero16
        return 0

    lax.fori_loop(0, NP // 16, _z1, 0)

    def _z2(i, _):
        zb[i // 8, pl.ds((i % 8) * 16, 16)] = zero16
        return 0

    lax.fori_loop(0, 32, _z2, 0)

    def _z3(i, _):
        eav[i // 8, pl.ds((i % 8) * 16, 16)] = zero16
        return 0

    lax.fori_loop(0, CH * 8, _z3, 0)
    for b in range(RPT // 4):
        pltpu.sync_copy(zb, asum_sh.at[pl.ds(row0 + b * 4, 4)])
    plsc.subcore_barrier()

    def _chunk(g, _):
        base = wid * ET + g * CH
        pltpu.sync_copy(src_h.at[pl.ds(base, CH)], srcv)
        pltpu.sync_copy(dst_h.at[pl.ds(base, CH)], dstv)
        pltpu.async_copy(ea_h.at[pl.ds(base, CH)], eav16, sem0).wait()

        def _vec(v, _):
            off = v * 16
            d16 = dstv[pl.ds(off, 16)]
            s16 = srcv[pl.ds(off, 16)]
            keep = s16 != d16
            dm = jnp.where(keep, d16, NP - 1)
            dmod[pl.ds(off, 16)] = dm
            plsc.addupdate_scatter(cnt_loc, [dm], ones16)
            return 0

        lax.fori_loop(0, CH // 16, _vec, 0)

        def _pe(e, _):
            eav[e, pl.ds(0, 16)] = eav16[e]
            return 0

        lax.fori_loop(0, CH, _pe, 0)
        pltpu.sync_copy(eav, asum_sh.at[dmod], add=True)
        return 0

    lax.fori_loop(0, NCH, _chunk, 0)
    pltpu.sync_copy(cnt_loc, cntp_h.at[wid])
    plsc.subcore_barrier()
    pltpu.sync_copy(asum_sh.at[pl.ds(row0, RPT)],
                    asump_h.at[cid, pl.ds(row0, RPT)])


@functools.lru_cache(maxsize=None)
def _p0_call():
    return functools.partial(
        pl.kernel,
        out_type=(jax.ShapeDtypeStruct((NW, NP), _f32),
                  jax.ShapeDtypeStruct((NC, NP, HID), _f32)),
        mesh=_sc_mesh(),
        compiler_params=pltpu.CompilerParams(needs_layout_passes=False),
        scratch_types=(
            pltpu.VMEM((CH,), _i32),
            pltpu.VMEM((CH,), _i32),
            pltpu.VMEM((CH,), _i32),
            pltpu.VMEM((CH, DE), _f32),
            pltpu.VMEM((CH, HID), _f32),
            pltpu.VMEM((NP,), _f32),
            pltpu.VMEM((4, HID), _f32),
            pltpu.VMEM_SHARED((NP, HID), _f32),
            pltpu.SemaphoreType.DMA,
        ),
    )(_p0_body)


SUP = 8 * H
NSUP = ETP // SUP


def _e1_body(xl_h, xr_h, ec_h, src_h, dst_h, att_h, outp_h, denp_h,
             xl_a, xr_a, ec_a, xl_c, xr_c, ec_c, src_sup, dst_sup,
             att_vm, den_loc, exb, zbuf, accum, sem_a, sem_c):
    cid = lax.axis_index("c")
    sid = lax.axis_index("s")
    wid = cid * NS + sid
    row0 = sid * RPT
    ebase = wid * ETP
    zero16 = jnp.zeros((16,), _f32)
    lane = lax.broadcasted_iota(_i32, (16,), 0)

    pltpu.sync_copy(att_h, att_vm)

    def _z1(i, _):
        den_loc[pl.ds(i * 16, 16)] = zero16
        return 0

    lax.fori_loop(0, NP // 16, _z1, 0)

    def _z2(i, _):
        zbuf[i // 8, pl.ds((i % 8) * 16, 16)] = zero16
        return 0

    lax.fori_loop(0, 128, _z2, 0)
    for b in range(RPT // 16):
        pltpu.sync_copy(zbuf, accum.at[pl.ds(row0 + b * 16, 16)])
    plsc.subcore_barrier()

    att_j = [att_vm[pl.ds(j * 16, 16)] for j in range(8)]

    def _fire(chunk, xb, rb, eb, sem):
        slot = (chunk // 8) % 2
        off = (chunk % 8) * H
        base = ebase + chunk * H
        pltpu.async_copy(xl_h.at[src_sup.at[slot, pl.ds(off, H)]], xb, sem)
        pltpu.async_copy(xr_h.at[dst_sup.at[slot, pl.ds(off, H)]], rb, sem)
        pltpu.async_copy(ec_h.at[pl.ds(base, H)], eb, sem)

    def _wait(xb, rb, eb, sem):
        pltpu.make_async_copy(ec_h.at[pl.ds(0, H)], xb, sem).wait()
        pltpu.make_async_copy(ec_h.at[pl.ds(0, H)], rb, sem).wait()
        pltpu.make_async_copy(ec_h.at[pl.ds(0, H)], eb, sem).wait()

    def _compute(xb, rb, eb, chunk):
        slot = (chunk // 8) % 2
        off = (chunk % 8) * H

        def _edge(i, _):
            for u in range(4):
                e = i * 4 + u
                acc = zero16
                xs = []
                for j in range(8):
                    sl = pl.ds(j * 16, 16)
                    xlv = xb[e, sl]
                    v = xlv + rb[e, sl] + eb[e, sl]
                    v = jnp.maximum(v, v * 0.2)
                    acc = acc + v * att_j[j]
                    xs.append(xlv)
                lg = jnp.sum(acc)
                iz = lane * 0
                sv = plsc.load_gather(src_sup, [iz + slot, iz + (off + e)])
                dv = plsc.load_gather(dst_sup, [iz + slot, iz + (off + e)])
                exv = jnp.exp(lax.broadcast(lg, (16,)))
                exv = exv * jnp.where(sv != dv, 1.0, 0.0)
                exb[pl.ds(e * 16, 16)] = exv
                for j in range(8):
                    xb[e, pl.ds(j * 16, 16)] = xs[j] * exv
            return 0

        lax.fori_loop(0, H // 4, _edge, 0)

        def _den(v, _):
            d16 = dst_sup[slot, pl.ds(off + v * 16, 16)]
            ex16 = plsc.load_gather(exb, [(lane + v * 16) * 16])
            plsc.addupdate_scatter(den_loc, [d16], ex16)
            return 0

        lax.fori_loop(0, H // 16, _den, 0)
        pltpu.sync_copy(xb, accum.at[dst_sup.at[slot, pl.ds(off, H)]],
                        add=True)

    for s in range(2):
        pltpu.sync_copy(src_h.at[pl.ds(ebase + s * SUP, SUP)],
                        src_sup.at[s])
        pltpu.sync_copy(dst_h.at[pl.ds(ebase + s * SUP, SUP)],
                        dst_sup.at[s])
    _fire(0, xl_a, xr_a, ec_a, sem_a)
    _fire(1, xl_c, xr_c, ec_c, sem_c)

    def _body(h, _):
        _wait(xl_a, xr_a, ec_a, sem_a)
        _compute(xl_a, xr_a, ec_a, 2 * h)

        @pl.when(h < NB - 1)
        def _():
            _fire(2 * h + 2, xl_a, xr_a, ec_a, sem_a)

        _wait(xl_c, xr_c, ec_c, sem_c)
        _compute(xl_c, xr_c, ec_c, 2 * h + 1)

        @pl.when(h < NB - 1)
        def _():
            _fire(2 * h + 3, xl_c, xr_c, ec_c, sem_c)

        @pl.when(jnp.logical_and(h % 4 == 3, h // 4 + 2 < NSUP))
        def _():
            s2 = h // 4 + 2
            slot2 = s2 % 2
            pltpu.sync_copy(src_h.at[pl.ds(ebase + s2 * SUP, SUP)],
                            src_sup.at[slot2])
            pltpu.sync_copy(dst_h.at[pl.ds(ebase + s2 * SUP, SUP)],
                            dst_sup.at[slot2])

        return 0

    lax.fori_loop(0, NB, _body, 0)
    pltpu.sync_copy(den_loc, denp_h.at[wid])
    plsc.subcore_barrier()
    pltpu.sync_copy(accum.at[pl.ds(row0, RPT)],
                    outp_h.at[cid, pl.ds(row0, RPT)])


@functools.lru_cache(maxsize=None)
def _e1_call():
    return functools.partial(
        pl.kernel,
        out_type=(jax.ShapeDtypeStruct((NC, NP, HID), _f32),
                  jax.ShapeDtypeStruct((NW, NP), _f32)),
        mesh=_sc_mesh(),
        compiler_params=pltpu.CompilerParams(needs_layout_passes=False),
        scratch_types=(
            pltpu.VMEM((H, HID), _f32),
            pltpu.VMEM((H, HID), _f32),
            pltpu.VMEM((H, HID), _f32),
            pltpu.VMEM((H, HID), _f32),
            pltpu.VMEM((H, HID), _f32),
            pltpu.VMEM((H, HID), _f32),
            pltpu.VMEM((2, SUP), _i32),
            pltpu.VMEM((2, SUP), _i32),
            pltpu.VMEM((HID,), _f32),
            pltpu.VMEM((NP,), _f32),
            pltpu.VMEM((H * 16,), _f32),
            pltpu.VMEM((16, HID), _f32),
            pltpu.VMEM_SHARED((NP, HID), _f32),
            pltpu.SemaphoreType.DMA,
            pltpu.SemaphoreType.DMA,
        ),
    )(_e1_body)


BA = 1280
BE = 2048


def _dot(a, b):
    return jnp.dot(a, b, precision=_HI, preferred_element_type=_f32)


def _gat_head(x, asum, cnt, Wl, bl, Wr, br, We, att, xl_ref, xr_ref, s_ref):
    xl = _dot(x, Wl) + bl
    xr = _dot(x, Wr) + br
    c = jnp.maximum(jnp.sum(cnt, axis=1, keepdims=True), 1.0)
    la = (asum[0, :, :DE] + asum[1, :, :DE]) / c
    lv = xl + xr + _dot(la, We)
    lv = jnp.maximum(lv, 0.2 * lv)
    s_ref[...] = jnp.sum(lv * att, axis=1, keepdims=True)
    xl_ref[...] = xl
    xr_ref[...] = xr


def _a0_kernel(x_ref, asum_ref, cnt_ref, Wl_ref, bl_ref, Wr_ref, br_ref,
               We_ref, att_ref, xl_ref, xr_ref, s_ref):
    _gat_head(x_ref[...], asum_ref[...], cnt_ref[...], Wl_ref[...],
              bl_ref[...], Wr_ref[...], br_ref[...], We_ref[...],
              att_ref[...], xl_ref, xr_ref, s_ref)


def _finish(outp, denp, xlp, sprev, bprev):
    es = jnp.exp(sprev)
    den = jnp.sum(denp, axis=1, keepdims=True) + es
    return jnp.tanh((outp[0] + outp[1] + es * xlp) / den + bprev)


def _al_kernel(outp_ref, denp_ref, xlp_ref, sprev_ref, bprev_ref,
               asum_ref, cnt_ref, Wl_ref, bl_ref, Wr_ref, br_ref, We_ref,
               att_ref, xl_ref, xr_ref, s_ref):
    h = _finish(outp_ref[...], denp_ref[...], xlp_ref[...], sprev_ref[...],
                bprev_ref[...])
    _gat_head(h, asum_ref[...], cnt_ref[...], Wl_ref[...], bl_ref[...],
              Wr_ref[...], br_ref[...], We_ref[...], att_ref[...],
              xl_ref, xr_ref, s_ref)


def _a5_kernel(outp_ref, denp_ref, xlp_ref, sprev_ref, bprev_ref,
               Wg1_ref, bg1_ref, Wg2_ref, bg2_ref, wg3_ref, bg3_ref,
               Wa1_ref, ba1_ref, Wa2_ref, ba2_ref, g_ref, t_ref):
    h = _finish(outp_ref[...], denp_ref[...], xlp_ref[...], sprev_ref[...],
                bprev_ref[...])
    g1 = jnp.tanh(_dot(h, Wg1_ref[...]) + bg1_ref[...])
    g2 = jnp.tanh(_dot(g1, Wg2_ref[...]) + bg2_ref[...])
    g_ref[...] = (jnp.sum(g2 * wg3_ref[...], axis=1, keepdims=True)
                  + bg3_ref[...])
    t1 = jnp.tanh(_dot(h, Wa1_ref[...]) + ba1_ref[...])
    t_ref[...] = jnp.tanh(_dot(t1, Wa2_ref[...]) + ba2_ref[...])


def _ec_kernel(ea_ref, We_ref, out_ref):
    out_ref[...] = _dot(ea_ref[...], We_ref[...])


def _pool_kernel(g_ref, t_ref, bidr_ref, bidc_ref, W1_ref, b1_ref, W2_ref,
                 b2_ref, W3_ref, b3_ref, out_ref):
    gr = g_ref[...]
    bidr = bidr_ref[...]
    gid = lax.broadcasted_iota(_i32, (G, NP), 0)
    oh = bidr == gid
    ohf = oh.astype(_f32)
    m = jnp.max(jnp.where(oh, gr, -1e30), axis=1, keepdims=True)
    mb = jnp.sum(ohf * m, axis=0, keepdims=True)
    ex = jnp.where(bidr < G, jnp.exp(gr - mb), 0.0)
    den = jnp.sum(ohf * ex, axis=1, keepdims=True)
    t = jnp.where(bidc_ref[...] < G, t_ref[...], 0.0)
    num = _dot(ohf * ex, t)
    pooled = num / jnp.maximum(den, 1e-30)
    o = jnp.tanh(_dot(pooled, W1_ref[...]) + b1_ref[...])
    o = jnp.tanh(_dot(o, W2_ref[...]) + b2_ref[...])
    o = jnp.tanh(_dot(o, W3_ref[...]) + b3_ref[...])
    out_ref[...] = o


def _full(shape):
    ndim = len(shape)
    return pl.BlockSpec(shape, lambda *_, _n=ndim: (0,) * _n)


def _a_call(kern, extra_specs, *args):
    grid = NP // BA
    out_shapes = [jax.ShapeDtypeStruct((NP, HID), _f32),
                  jax.ShapeDtypeStruct((NP, HID), _f32),
                  jax.ShapeDtypeStruct((NP, 1), _f32)]
    out_specs = [pl.BlockSpec((BA, HID), lambda i: (i, 0)),
                 pl.BlockSpec((BA, HID), lambda i: (i, 0)),
                 pl.BlockSpec((BA, 1), lambda i: (i, 0))]
    return pl.pallas_call(
        kern, grid=(grid,), in_specs=list(extra_specs),
        out_specs=out_specs, out_shape=out_shapes,
    )(*args)


_SPEC_X = pl.BlockSpec((BA, HID), lambda i: (i, 0))
_SPEC_S = pl.BlockSpec((BA, 1), lambda i: (i, 0))
_SPEC_ASUM = pl.BlockSpec((NC, BA, HID), lambda i: (0, i, 0))
_SPEC_CNT = pl.BlockSpec((BA, NW), lambda i: (i, 0))
_SPEC_DEN = pl.BlockSpec((BA, NW), lambda i: (i, 0))
_SPEC_OUTP = pl.BlockSpec((NC, BA, HID), lambda i: (0, i, 0))
_SPEC_W = _full((HID, HID))
_SPEC_B = _full((HID,))
_SPEC_WE = _full((DE, HID))


def kernel(x, edge_index, edge_attr, batch_ids, params):
    src = edge_index[0]
    dst = edge_index[1]
    srcp = jnp.pad(src, (0, EP - E), constant_values=NP - 1)
    dstp = jnp.pad(dst, (0, EP - E), constant_values=NP - 1)
    eap = jnp.pad(edge_attr, ((0, EP - E), (0, 0)))
    xp = jnp.pad(x, ((0, NP - N), (0, 0)))
    bidp = jnp.pad(batch_ids, (0, NP - N), constant_values=G)

    cnt_p, asum_p = _p0_call()(src, dst, edge_attr)
    cnt_p = jnp.transpose(cnt_p)

    gps = params["gat"]
    outp = None
    denp = None
    xl_prev = None
    s_prev = None
    for l, p in enumerate(gps):
        if l == 0:
            xl, xr, s = _a_call(
                _a0_kernel,
                [_SPEC_X, _SPEC_ASUM, _SPEC_CNT, _SPEC_W, _SPEC_B, _SPEC_W,
                 _SPEC_B, _SPEC_WE, _SPEC_B],
                xp, asum_p, cnt_p, p["Wl"], p["bl"], p["Wr"], p["br"],
                p["We"], p["att"])
        else:
            bprev = gps[l - 1]["bias"]
            xl, xr, s = _a_call(
                _al_kernel,
                [_SPEC_OUTP, _SPEC_DEN, _SPEC_X, _SPEC_S, _SPEC_B,
                 _SPEC_ASUM, _SPEC_CNT, _SPEC_W, _SPEC_B, _SPEC_W, _SPEC_B,
                 _SPEC_WE, _SPEC_B],
                outp, denp, xl_prev, s_prev, bprev, asum_p, cnt_p,
                p["Wl"], p["bl"], p["Wr"], p["br"], p["We"], p["att"])

        ec = pl.pallas_call(
            _ec_kernel, grid=(EP // BE,),
            in_specs=[pl.BlockSpec((BE, DE), lambda i: (i, 0)), _SPEC_WE],
            out_specs=pl.BlockSpec((BE, HID), lambda i: (i, 0)),
            out_shape=jax.ShapeDtypeStruct((EP, HID), _f32),
        )(eap, p["We"])

        outp, denp = _e1_call()(xl, xr, ec, srcp, dstp, p["att"])
        denp = jnp.transpose(denp)
        xl_prev = xl
        s_prev = s

    (Wg1, bg1), (Wg2, bg2), (Wg3, bg3) = params["gate_nn"]
    (Wa1, ba1), (Wa2, ba2) = params["att_nn"]
    g, t = pl.pallas_call(
        _a5_kernel, grid=(NP // BA,),
        in_specs=[_SPEC_OUTP, _SPEC_DEN, _SPEC_X, _SPEC_S, _SPEC_B,
                  _SPEC_W, _SPEC_B, _SPEC_W, _SPEC_B, _SPEC_B, _full((1,)),
                  _SPEC_W, _SPEC_B, _SPEC_W, _SPEC_B],
        out_specs=[pl.BlockSpec((BA, 1), lambda i: (i, 0)), _SPEC_X],
        out_shape=[jax.ShapeDtypeStruct((NP, 1), _f32),
                   jax.ShapeDtypeStruct((NP, HID), _f32)],
    )(outp, denp, xl_prev, s_prev, gps[4]["bias"], Wg1, bg1, Wg2, bg2,
      Wg3[:, 0], bg3, Wa1, ba1, Wa2, ba2)

    (W1, b1), (W2, b2), (W3, b3) = params["fc"]
    out = pl.pallas_call(
        _pool_kernel,
        in_specs=[_full((1, NP)), _full((NP, HID)), _full((1, NP)),
                  _full((NP, 1)), _SPEC_W, _SPEC_B, _SPEC_W, _SPEC_B,
                  _full((HID, 1)), _full((1,))],
        out_specs=_full((G, 1)),
        out_shape=jax.ShapeDtypeStruct((G, 1), _f32),
    )(jnp.reshape(g, (1, NP)), t, jnp.reshape(bidp, (1, NP)),
      jnp.reshape(bidp, (NP, 1)), W1, b1, W2, b2, W3, b3)
    return out


# speedup vs baseline: 1.0222x; 1.0222x over previous
import functools

import jax
import jax.numpy as jnp
from jax import lax
from jax.experimental import pallas as pl
from jax.experimental.pallas import tpu as pltpu
from jax.experimental.pallas import tpu_sc as plsc

N = 10000
E = 320000
HID = 128
DE = 16
G = 16

NP = 10240
NC = 2
NS = 16
NW = NC * NS
ET = E // NW
CH = 80
NCH = ET // CH
RPT = NP // NS

EP = 327680
ETP = EP // NW
H = 32
NB = ETP // (2 * H)

_f32 = jnp.float32
_i32 = jnp.int32
_HI = jax.lax.Precision.HIGHEST


def _sc_mesh():
    return plsc.VectorSubcoreMesh(core_axis_name="c", subcore_axis_name="s")


def _p0_body(src_h, dst_h, ea_h, cntp_h, asump_h,
             srcv, dstv, dmod, eav16, eav, cnt_loc, zb, asum_sh, sem0):
    cid = lax.axis_index("c")
    sid = lax.axis_index("s")
    wid = cid * NS + sid
    row0 = sid * RPT
    zero16 = jnp.zeros((16,), _f32)
    ones16 = zero16 + 1.0

    def _z1(i, _):
        cnt_loc[pl.ds(i * 16, 16)] = zero16
        return 0

    lax.fori_loop(0, NP // 16, _z1, 0)

    def _z2(i, _):
        zb[i // 8, pl.ds((i % 8) * 16, 16)] = zero16
        return 0

    lax.fori_loop(0, 32, _z2, 0)

    def _z3(i, _):
        eav[i // 8, pl.ds((i % 8) * 16, 16)] = zero16
        return 0

    lax.fori_loop(0, CH * 8, _z3, 0)
    for b in range(RPT // 4):
        pltpu.sync_copy(zb, asum_sh.at[pl.ds(row0 + b * 4, 4)])
    plsc.subcore_barrier()

    def _sup(g, _):
        base = wid * ET + g * (5 * CH)
        pltpu.sync_copy(src_h.at[pl.ds(base, 5 * CH)], srcv)
        pltpu.sync_copy(dst_h.at[pl.ds(base, 5 * CH)], dstv)
        pltpu.async_copy(ea_h.at[pl.ds(base * DE, 5 * CH * DE)], eav16,
                         sem0).wait()

        def _chunk(q, _):
            coff = q * CH

            def _vec(v, _):
                off = coff + v * 16
                d16 = dstv[pl.ds(off, 16)]
                s16 = srcv[pl.ds(off, 16)]
                keep = s16 != d16
                dm = jnp.where(keep, d16, NP - 1)
                dmod[pl.ds(v * 16, 16)] = dm
                plsc.addupdate_scatter(cnt_loc, [dm], ones16)
                return 0

            lax.fori_loop(0, CH // 16, _vec, 0)

            def _pe(e, _):
                eav[e, pl.ds(0, 16)] = eav16[pl.ds((coff + e) * DE, 16)]
                return 0

            lax.fori_loop(0, CH, _pe, 0)
            pltpu.sync_copy(eav, asum_sh.at[dmod], add=True)
            return 0

        lax.fori_loop(0, 5, _chunk, 0)
        return 0

    lax.fori_loop(0, NCH // 5, _sup, 0)
    pltpu.sync_copy(cnt_loc, cntp_h.at[wid])
    plsc.subcore_barrier()
    pltpu.sync_copy(asum_sh.at[pl.ds(row0, RPT)],
                    asump_h.at[cid, pl.ds(row0, RPT)])


@functools.lru_cache(maxsize=None)
def _p0_call():
    return functools.partial(
        pl.kernel,
        out_type=(jax.ShapeDtypeStruct((NW, NP), _f32),
                  jax.ShapeDtypeStruct((NC, NP, HID), _f32)),
        mesh=_sc_mesh(),
        compiler_params=pltpu.CompilerParams(needs_layout_passes=False),
        scratch_types=(
            pltpu.VMEM((5 * CH,), _i32),
            pltpu.VMEM((5 * CH,), _i32),
            pltpu.VMEM((CH,), _i32),
            pltpu.VMEM((5 * CH * DE,), _f32),
            pltpu.VMEM((CH, HID), _f32),
            pltpu.VMEM((NP,), _f32),
            pltpu.VMEM((4, HID), _f32),
            pltpu.VMEM_SHARED((NP, HID), _f32),
            pltpu.SemaphoreType.DMA,
        ),
    )(_p0_body)


SUP = 8 * H
NSUP = ETP // SUP


def _e1_body(xl_h, xr_h, ec_h, src_h, dst_h, att_h, outp_h, denp_h,
             xl_a, xr_a, ec_a, xl_c, xr_c, ec_c, src_sup, dst_sup,
             att_vm, den_loc, exb, zbuf, accum, sem_a, sem_c):
    cid = lax.axis_index("c")
    sid = lax.axis_index("s")
    wid = cid * NS + sid
    row0 = sid * RPT
    ebase = wid * ETP
    zero16 = jnp.zeros((16,), _f32)
    lane = lax.broadcasted_iota(_i32, (16,), 0)

    pltpu.sync_copy(att_h, att_vm)

    def _z1(i, _):
        den_loc[pl.ds(i * 16, 16)] = zero16
        return 0

    lax.fori_loop(0, NP // 16, _z1, 0)

    def _z2(i, _):
        zbuf[i // 8, pl.ds((i % 8) * 16, 16)] = zero16
        return 0

    lax.fori_loop(0, 128, _z2, 0)
    for b in range(RPT // 16):
        pltpu.sync_copy(zbuf, accum.at[pl.ds(row0 + b * 16, 16)])
    plsc.subcore_barrier()

    att_j = [att_vm[pl.ds(j * 16, 16)] for j in range(8)]

    def _fire(chunk, xb, rb, eb, sem):
        slot = (chunk // 8) % 2
        off = (chunk % 8) * H
        base = ebase + chunk * H
        pltpu.async_copy(xl_h.at[src_sup.at[slot, pl.ds(off, H)]], xb, sem)
        pltpu.async_copy(xr_h.at[dst_sup.at[slot, pl.ds(off, H)]], rb, sem)
        pltpu.async_copy(ec_h.at[pl.ds(base, H)], eb, sem)

    def _wait(xb, rb, eb, sem):
        pltpu.make_async_copy(ec_h.at[pl.ds(0, H)], xb, sem).wait()
        pltpu.make_async_copy(ec_h.at[pl.ds(0, H)], rb, sem).wait()
        pltpu.make_async_copy(ec_h.at[pl.ds(0, H)], eb, sem).wait()

    def _compute(xb, rb, eb, chunk):
        slot = (chunk // 8) % 2
        off = (chunk % 8) * H

        def _edge(i, _):
            for u in range(4):
                e = i * 4 + u
                acc = zero16
                xs = []
                for j in range(8):
                    sl = pl.ds(j * 16, 16)
                    xlv = xb[e, sl]
                    v = xlv + rb[e, sl] + eb[e, sl]
                    v = jnp.maximum(v, v * 0.2)
                    acc = acc + v * att_j[j]
                    xs.append(xlv)
                lg = jnp.sum(acc)
                iz = lane * 0
                sv = plsc.load_gather(src_sup, [iz + slot, iz + (off + e)])
                dv = plsc.load_gather(dst_sup, [iz + slot, iz + (off + e)])
                exv = jnp.exp(lax.broadcast(lg, (16,)))
                exv = exv * jnp.where(sv != dv, 1.0, 0.0)
                exb[pl.ds(e * 16, 16)] = exv
                for j in range(8):
                    xb[e, pl.ds(j * 16, 16)] = xs[j] * exv
            return 0

        lax.fori_loop(0, H // 4, _edge, 0)

        def _den(v, _):
            d16 = dst_sup[slot, pl.ds(off + v * 16, 16)]
            ex16 = plsc.load_gather(exb, [(lane + v * 16) * 16])
            plsc.addupdate_scatter(den_loc, [d16], ex16)
            return 0

        lax.fori_loop(0, H // 16, _den, 0)
        pltpu.sync_copy(xb, accum.at[dst_sup.at[slot, pl.ds(off, H)]],
                        add=True)

    for s in range(2):
        pltpu.sync_copy(src_h.at[pl.ds(ebase + s * SUP, SUP)],
                        src_sup.at[s])
        pltpu.sync_copy(dst_h.at[pl.ds(ebase + s * SUP, SUP)],
                        dst_sup.at[s])
    _fire(0, xl_a, xr_a, ec_a, sem_a)
    _fire(1, xl_c, xr_c, ec_c, sem_c)

    def _body(h, _):
        _wait(xl_a, xr_a, ec_a, sem_a)
        _compute(xl_a, xr_a, ec_a, 2 * h)

        @pl.when(h < NB - 1)
        def _():
            _fire(2 * h + 2, xl_a, xr_a, ec_a, sem_a)

        _wait(xl_c, xr_c, ec_c, sem_c)
        _compute(xl_c, xr_c, ec_c, 2 * h + 1)

        @pl.when(h < NB - 1)
        def _():
            _fire(2 * h + 3, xl_c, xr_c, ec_c, sem_c)

        @pl.when(jnp.logical_and(h % 4 == 3, h // 4 + 2 < NSUP))
        def _():
            s2 = h // 4 + 2
            slot2 = s2 % 2
            pltpu.sync_copy(src_h.at[pl.ds(ebase + s2 * SUP, SUP)],
                            src_sup.at[slot2])
            pltpu.sync_copy(dst_h.at[pl.ds(ebase + s2 * SUP, SUP)],
                            dst_sup.at[slot2])

        return 0

    lax.fori_loop(0, NB, _body, 0)
    pltpu.sync_copy(den_loc, denp_h.at[wid])
    plsc.subcore_barrier()
    pltpu.sync_copy(accum.at[pl.ds(row0, RPT)],
                    outp_h.at[cid, pl.ds(row0, RPT)])


@functools.lru_cache(maxsize=None)
def _e1_call():
    return functools.partial(
        pl.kernel,
        out_type=(jax.ShapeDtypeStruct((NC, NP, HID), _f32),
                  jax.ShapeDtypeStruct((NW, NP), _f32)),
        mesh=_sc_mesh(),
        compiler_params=pltpu.CompilerParams(needs_layout_passes=False),
        scratch_types=(
            pltpu.VMEM((H, HID), _f32),
            pltpu.VMEM((H, HID), _f32),
            pltpu.VMEM((H, HID), _f32),
            pltpu.VMEM((H, HID), _f32),
            pltpu.VMEM((H, HID), _f32),
            pltpu.VMEM((H, HID), _f32),
            pltpu.VMEM((2, SUP), _i32),
            pltpu.VMEM((2, SUP), _i32),
            pltpu.VMEM((HID,), _f32),
            pltpu.VMEM((NP,), _f32),
            pltpu.VMEM((H * 16,), _f32),
            pltpu.VMEM((16, HID), _f32),
            pltpu.VMEM_SHARED((NP, HID), _f32),
            pltpu.SemaphoreType.DMA,
            pltpu.SemaphoreType.DMA,
        ),
    )(_e1_body)


BA = 1280
BE = 2048


def _dot(a, b):
    return jnp.dot(a, b, precision=_HI, preferred_element_type=_f32)


def _gat_head(x, asum, cnt, Wl, bl, Wr, br, We, att, xl_ref, xr_ref, s_ref):
    xl = _dot(x, Wl) + bl
    xr = _dot(x, Wr) + br
    c = jnp.maximum(jnp.sum(cnt, axis=1, keepdims=True), 1.0)
    la = (asum[0, :, :DE] + asum[1, :, :DE]) / c
    lv = xl + xr + _dot(la, We)
    lv = jnp.maximum(lv, 0.2 * lv)
    s_ref[...] = jnp.sum(lv * att, axis=1, keepdims=True)
    xl_ref[...] = xl
    xr_ref[...] = xr


def _a0_kernel(x_ref, asum_ref, cnt_ref, Wl_ref, bl_ref, Wr_ref, br_ref,
               We_ref, att_ref, xl_ref, xr_ref, s_ref):
    _gat_head(x_ref[...], asum_ref[...], cnt_ref[...], Wl_ref[...],
              bl_ref[...], Wr_ref[...], br_ref[...], We_ref[...],
              att_ref[...], xl_ref, xr_ref, s_ref)


def _finish(outp, denp, xlp, sprev, bprev):
    es = jnp.exp(sprev)
    den = jnp.sum(denp, axis=1, keepdims=True) + es
    return jnp.tanh((outp[0] + outp[1] + es * xlp) / den + bprev)


def _al_kernel(outp_ref, denp_ref, xlp_ref, sprev_ref, bprev_ref,
               asum_ref, cnt_ref, Wl_ref, bl_ref, Wr_ref, br_ref, We_ref,
               att_ref, xl_ref, xr_ref, s_ref):
    h = _finish(outp_ref[...], denp_ref[...], xlp_ref[...], sprev_ref[...],
                bprev_ref[...])
    _gat_head(h, asum_ref[...], cnt_ref[...], Wl_ref[...], bl_ref[...],
              Wr_ref[...], br_ref[...], We_ref[...], att_ref[...],
              xl_ref, xr_ref, s_ref)


def _a5_kernel(outp_ref, denp_ref, xlp_ref, sprev_ref, bprev_ref,
               Wg1_ref, bg1_ref, Wg2_ref, bg2_ref, wg3_ref, bg3_ref,
               Wa1_ref, ba1_ref, Wa2_ref, ba2_ref, g_ref, t_ref):
    h = _finish(outp_ref[...], denp_ref[...], xlp_ref[...], sprev_ref[...],
                bprev_ref[...])
    g1 = jnp.tanh(_dot(h, Wg1_ref[...]) + bg1_ref[...])
    g2 = jnp.tanh(_dot(g1, Wg2_ref[...]) + bg2_ref[...])
    g_ref[...] = (jnp.sum(g2 * wg3_ref[...], axis=1, keepdims=True)
                  + bg3_ref[...])
    t1 = jnp.tanh(_dot(h, Wa1_ref[...]) + ba1_ref[...])
    t_ref[...] = jnp.tanh(_dot(t1, Wa2_ref[...]) + ba2_ref[...])


def _ec_kernel(ea_ref, We_ref, out_ref):
    out_ref[...] = _dot(ea_ref[...], We_ref[...])


def _pool_kernel(g_ref, t_ref, bidr_ref, bidc_ref, W1_ref, b1_ref, W2_ref,
                 b2_ref, W3_ref, b3_ref, out_ref):
    gr = g_ref[...]
    bidr = bidr_ref[...]
    gid = lax.broadcasted_iota(_i32, (G, NP), 0)
    oh = bidr == gid
    ohf = oh.astype(_f32)
    m = jnp.max(jnp.where(oh, gr, -1e30), axis=1, keepdims=True)
    mb = jnp.sum(ohf * m, axis=0, keepdims=True)
    ex = jnp.where(bidr < G, jnp.exp(gr - mb), 0.0)
    den = jnp.sum(ohf * ex, axis=1, keepdims=True)
    t = jnp.where(bidc_ref[...] < G, t_ref[...], 0.0)
    num = _dot(ohf * ex, t)
    pooled = num / jnp.maximum(den, 1e-30)
    o = jnp.tanh(_dot(pooled, W1_ref[...]) + b1_ref[...])
    o = jnp.tanh(_dot(o, W2_ref[...]) + b2_ref[...])
    o = jnp.tanh(_dot(o, W3_ref[...]) + b3_ref[...])
    out_ref[...] = o


def _full(shape):
    ndim = len(shape)
    return pl.BlockSpec(shape, lambda *_, _n=ndim: (0,) * _n)


def _a_call(kern, extra_specs, *args):
    grid = NP // BA
    out_shapes = [jax.ShapeDtypeStruct((NP, HID), _f32),
                  jax.ShapeDtypeStruct((NP, HID), _f32),
                  jax.ShapeDtypeStruct((NP, 1), _f32)]
    out_specs = [pl.BlockSpec((BA, HID), lambda i: (i, 0)),
                 pl.BlockSpec((BA, HID), lambda i: (i, 0)),
                 pl.BlockSpec((BA, 1), lambda i: (i, 0))]
    return pl.pallas_call(
        kern, grid=(grid,), in_specs=list(extra_specs),
        out_specs=out_specs, out_shape=out_shapes,
    )(*args)


_SPEC_X = pl.BlockSpec((BA, HID), lambda i: (i, 0))
_SPEC_S = pl.BlockSpec((BA, 1), lambda i: (i, 0))
_SPEC_ASUM = pl.BlockSpec((NC, BA, HID), lambda i: (0, i, 0))
_SPEC_CNT = pl.BlockSpec((BA, NW), lambda i: (i, 0))
_SPEC_DEN = pl.BlockSpec((BA, NW), lambda i: (i, 0))
_SPEC_OUTP = pl.BlockSpec((NC, BA, HID), lambda i: (0, i, 0))
_SPEC_W = _full((HID, HID))
_SPEC_B = _full((HID,))
_SPEC_WE = _full((DE, HID))


def kernel(x, edge_index, edge_attr, batch_ids, params):
    src = edge_index[0]
    dst = edge_index[1]
    srcp = jnp.pad(src, (0, EP - E), constant_values=NP - 1)
    dstp = jnp.pad(dst, (0, EP - E), constant_values=NP - 1)
    eap = jnp.pad(edge_attr, ((0, EP - E), (0, 0)))
    xp = jnp.pad(x, ((0, NP - N), (0, 0)))
    bidp = jnp.pad(batch_ids, (0, NP - N), constant_values=G)

    cnt_p, asum_p = _p0_call()(src, dst, jnp.reshape(edge_attr, (E * DE,)))
    cnt_p = jnp.transpose(cnt_p)

    gps = params["gat"]
    outp = None
    denp = None
    xl_prev = None
    s_prev = None
    for l, p in enumerate(gps):
        if l == 0:
            xl, xr, s = _a_call(
                _a0_kernel,
                [_SPEC_X, _SPEC_ASUM, _SPEC_CNT, _SPEC_W, _SPEC_B, _SPEC_W,
                 _SPEC_B, _SPEC_WE, _SPEC_B],
                xp, asum_p, cnt_p, p["Wl"], p["bl"], p["Wr"], p["br"],
                p["We"], p["att"])
        else:
            bprev = gps[l - 1]["bias"]
            xl, xr, s = _a_call(
                _al_kernel,
                [_SPEC_OUTP, _SPEC_DEN, _SPEC_X, _SPEC_S, _SPEC_B,
                 _SPEC_ASUM, _SPEC_CNT, _SPEC_W, _SPEC_B, _SPEC_W, _SPEC_B,
                 _SPEC_WE, _SPEC_B],
                outp, denp, xl_prev, s_prev, bprev, asum_p, cnt_p,
                p["Wl"], p["bl"], p["Wr"], p["br"], p["We"], p["att"])

        ec = pl.pallas_call(
            _ec_kernel, grid=(EP // BE,),
            in_specs=[pl.BlockSpec((BE, DE), lambda i: (i, 0)), _SPEC_WE],
            out_specs=pl.BlockSpec((BE, HID), lambda i: (i, 0)),
            out_shape=jax.ShapeDtypeStruct((EP, HID), _f32),
        )(eap, p["We"])

        outp, denp = _e1_call()(xl, xr, ec, srcp, dstp, p["att"])
        denp = jnp.transpose(denp)
        xl_prev = xl
        s_prev = s

    (Wg1, bg1), (Wg2, bg2), (Wg3, bg3) = params["gate_nn"]
    (Wa1, ba1), (Wa2, ba2) = params["att_nn"]
    g, t = pl.pallas_call(
        _a5_kernel, grid=(NP // BA,),
        in_specs=[_SPEC_OUTP, _SPEC_DEN, _SPEC_X, _SPEC_S, _SPEC_B,
                  _SPEC_W, _SPEC_B, _SPEC_W, _SPEC_B, _SPEC_B, _full((1,)),
                  _SPEC_W, _SPEC_B, _SPEC_W, _SPEC_B],
        out_specs=[pl.BlockSpec((BA, 1), lambda i: (i, 0)), _SPEC_X],
        out_shape=[jax.ShapeDtypeStruct((NP, 1), _f32),
                   jax.ShapeDtypeStruct((NP, HID), _f32)],
    )(outp, denp, xl_prev, s_prev, gps[4]["bias"], Wg1, bg1, Wg2, bg2,
      Wg3[:, 0], bg3, Wa1, ba1, Wa2, ba2)

    (W1, b1), (W2, b2), (W3, b3) = params["fc"]
    out = pl.pallas_call(
        _pool_kernel,
        in_specs=[_full((1, NP)), _full((NP, HID)), _full((1, NP)),
                  _full((NP, 1)), _SPEC_W, _SPEC_B, _SPEC_W, _SPEC_B,
                  _full((HID, 1)), _full((1,))],
        out_specs=_full((G, 1)),
        out_shape=jax.ShapeDtypeStruct((G, 1), _f32),
    )(jnp.reshape(g, (1, NP)), t, jnp.reshape(bidp, (1, NP)),
      jnp.reshape(bidp, (NP, 1)), W1, b1, W2, b2, W3, b3)
    return out


# speedup vs baseline: 1.0656x; 1.0424x over previous
import functools

import jax
import jax.numpy as jnp
from jax import lax
from jax.experimental import pallas as pl
from jax.experimental.pallas import tpu as pltpu
from jax.experimental.pallas import tpu_sc as plsc

N = 10000
E = 320000
HID = 128
DE = 16
G = 16

NP = 10240
NC = 2
NS = 16
NW = NC * NS
ET = E // NW
CH = 80
NCH = ET // CH
RPT = NP // NS

EP = 327680
ETP = EP // NW
H = 32
NB = ETP // (2 * H)

_f32 = jnp.float32
_i32 = jnp.int32
_HI = jax.lax.Precision.HIGHEST


def _sc_mesh():
    return plsc.VectorSubcoreMesh(core_axis_name="c", subcore_axis_name="s")


def _p0_body(src_h, dst_h, ea_h, cntp_h, asump_h,
             srcv, dstv, dmod, eav16, eav, cnt_loc, zb, asum_sh, sem0):
    cid = lax.axis_index("c")
    sid = lax.axis_index("s")
    wid = cid * NS + sid
    row0 = sid * RPT
    zero16 = jnp.zeros((16,), _f32)
    ones16 = zero16 + 1.0

    def _z1(i, _):
        cnt_loc[pl.ds(i * 16, 16)] = zero16
        return 0

    lax.fori_loop(0, NP // 16, _z1, 0)

    def _z2(i, _):
        zb[i // 8, pl.ds((i % 8) * 16, 16)] = zero16
        return 0

    lax.fori_loop(0, 32, _z2, 0)

    def _z3(i, _):
        eav[i // 8, pl.ds((i % 8) * 16, 16)] = zero16
        return 0

    lax.fori_loop(0, CH * 8, _z3, 0)
    for b in range(RPT // 4):
        pltpu.sync_copy(zb, asum_sh.at[pl.ds(row0 + b * 4, 4)])
    plsc.subcore_barrier()

    def _sup(g, _):
        base = wid * ET + g * (5 * CH)
        pltpu.sync_copy(src_h.at[pl.ds(base, 5 * CH)], srcv)
        pltpu.sync_copy(dst_h.at[pl.ds(base, 5 * CH)], dstv)
        pltpu.async_copy(ea_h.at[pl.ds(base * DE, 5 * CH * DE)], eav16,
                         sem0).wait()

        def _chunk(q, _):
            coff = q * CH

            def _vec(v, _):
                off = coff + v * 16
                d16 = dstv[pl.ds(off, 16)]
                s16 = srcv[pl.ds(off, 16)]
                keep = s16 != d16
                dm = jnp.where(keep, d16, NP - 1)
                dmod[pl.ds(v * 16, 16)] = dm
                plsc.addupdate_scatter(cnt_loc, [dm], ones16)
                return 0

            lax.fori_loop(0, CH // 16, _vec, 0)

            def _pe(e, _):
                eav[e, pl.ds(0, 16)] = eav16[pl.ds((coff + e) * DE, 16)]
                return 0

            lax.fori_loop(0, CH, _pe, 0)
            pltpu.sync_copy(eav, asum_sh.at[dmod], add=True)
            return 0

        lax.fori_loop(0, 5, _chunk, 0)
        return 0

    lax.fori_loop(0, NCH // 5, _sup, 0)
    pltpu.sync_copy(cnt_loc, cntp_h.at[wid])
    plsc.subcore_barrier()
    pltpu.sync_copy(asum_sh.at[pl.ds(row0, RPT)],
                    asump_h.at[cid, pl.ds(row0, RPT)])


@functools.lru_cache(maxsize=None)
def _p0_call():
    return functools.partial(
        pl.kernel,
        out_type=(jax.ShapeDtypeStruct((NW, NP), _f32),
                  jax.ShapeDtypeStruct((NC, NP, HID), _f32)),
        mesh=_sc_mesh(),
        compiler_params=pltpu.CompilerParams(needs_layout_passes=False),
        scratch_types=(
            pltpu.VMEM((5 * CH,), _i32),
            pltpu.VMEM((5 * CH,), _i32),
            pltpu.VMEM((CH,), _i32),
            pltpu.VMEM((5 * CH * DE,), _f32),
            pltpu.VMEM((CH, HID), _f32),
            pltpu.VMEM((NP,), _f32),
            pltpu.VMEM((4, HID), _f32),
            pltpu.VMEM_SHARED((NP, HID), _f32),
            pltpu.SemaphoreType.DMA,
        ),
    )(_p0_body)


SUP = 8 * H
NSUP = ETP // SUP


def _e1_body(xl_h, xr_h, ec_h, src_h, dst_h, att_h, outp_h, denp_h,
             xl_a, xr_a, ec_a, xl_c, xr_c, ec_c, src_sup, dst_sup,
             att_vm, den_loc, exb, zbuf, accum, sem_a, sem_c):
    cid = lax.axis_index("c")
    sid = lax.axis_index("s")
    wid = cid * NS + sid
    row0 = sid * RPT
    ebase = wid * ETP
    zero16 = jnp.zeros((16,), _f32)
    lane = lax.broadcasted_iota(_i32, (16,), 0)

    pltpu.sync_copy(att_h, att_vm)

    def _z1(i, _):
        den_loc[pl.ds(i * 16, 16)] = zero16
        return 0

    lax.fori_loop(0, NP // 16, _z1, 0)

    def _z2(i, _):
        zbuf[i // 8, pl.ds((i % 8) * 16, 16)] = zero16
        return 0

    lax.fori_loop(0, 128, _z2, 0)
    for b in range(RPT // 16):
        pltpu.sync_copy(zbuf, accum.at[pl.ds(row0 + b * 16, 16)])
    plsc.subcore_barrier()

    att_j = [att_vm[pl.ds(j * 16, 16)] for j in range(8)]

    def _fire(chunk, xb, rb, eb, sem):
        slot = (chunk // 8) % 2
        off = (chunk % 8) * H
        base = ebase + chunk * H
        pltpu.async_copy(xl_h.at[src_sup.at[slot, pl.ds(off, H)]], xb, sem)
        pltpu.async_copy(xr_h.at[dst_sup.at[slot, pl.ds(off, H)]], rb, sem)
        pltpu.async_copy(ec_h.at[pl.ds(base, H)], eb, sem)

    def _wait(xb, rb, eb, sem):
        pltpu.make_async_copy(ec_h.at[pl.ds(0, H)], xb, sem).wait()
        pltpu.make_async_copy(ec_h.at[pl.ds(0, H)], rb, sem).wait()
        pltpu.make_async_copy(ec_h.at[pl.ds(0, H)], eb, sem).wait()

    def _compute(xb, rb, eb, chunk):
        slot = (chunk // 8) % 2
        off = (chunk % 8) * H

        def _edge(i, _):
            for u in range(4):
                e = i * 4 + u
                acc = zero16
                xs = []
                for j in range(8):
                    sl = pl.ds(j * 16, 16)
                    xlv = xb[e, sl]
                    v = xlv + rb[e, sl] + eb[e, sl]
                    v = jnp.maximum(v, v * 0.2)
                    acc = acc + v * att_j[j]
                    xs.append(xlv)
                lg = jnp.sum(acc)
                iz = lane * 0
                sv = plsc.load_gather(src_sup, [iz + slot, iz + (off + e)])
                dv = plsc.load_gather(dst_sup, [iz + slot, iz + (off + e)])
                exv = jnp.exp(lax.broadcast(lg, (16,)))
                exv = exv * jnp.where(sv != dv, 1.0, 0.0)
                exb[pl.ds(e * 16, 16)] = exv
                for j in range(8):
                    xb[e, pl.ds(j * 16, 16)] = xs[j] * exv
            return 0

        lax.fori_loop(0, H // 4, _edge, 0)

        def _den(v, _):
            d16 = dst_sup[slot, pl.ds(off + v * 16, 16)]
            ex16 = plsc.load_gather(exb, [(lane + v * 16) * 16])
            plsc.addupdate_scatter(den_loc, [d16], ex16)
            return 0

        lax.fori_loop(0, H // 16, _den, 0)
        pltpu.sync_copy(xb, accum.at[dst_sup.at[slot, pl.ds(off, H)]],
                        add=True)

    for s in range(2):
        pltpu.sync_copy(src_h.at[pl.ds(ebase + s * SUP, SUP)],
                        src_sup.at[s])
        pltpu.sync_copy(dst_h.at[pl.ds(ebase + s * SUP, SUP)],
                        dst_sup.at[s])
    _fire(0, xl_a, xr_a, ec_a, sem_a)
    _fire(1, xl_c, xr_c, ec_c, sem_c)

    def _body(h, _):
        _wait(xl_a, xr_a, ec_a, sem_a)
        _compute(xl_a, xr_a, ec_a, 2 * h)

        @pl.when(h < NB - 1)
        def _():
            _fire(2 * h + 2, xl_a, xr_a, ec_a, sem_a)

        _wait(xl_c, xr_c, ec_c, sem_c)
        _compute(xl_c, xr_c, ec_c, 2 * h + 1)

        @pl.when(h < NB - 1)
        def _():
            _fire(2 * h + 3, xl_c, xr_c, ec_c, sem_c)

        @pl.when(jnp.logical_and(h % 4 == 3, h // 4 + 2 < NSUP))
        def _():
            s2 = h // 4 + 2
            slot2 = s2 % 2
            pltpu.sync_copy(src_h.at[pl.ds(ebase + s2 * SUP, SUP)],
                            src_sup.at[slot2])
            pltpu.sync_copy(dst_h.at[pl.ds(ebase + s2 * SUP, SUP)],
                            dst_sup.at[slot2])

        return 0

    lax.fori_loop(0, NB, _body, 0)
    pltpu.sync_copy(den_loc, denp_h.at[wid])
    plsc.subcore_barrier()
    pltpu.sync_copy(accum.at[pl.ds(row0, RPT)],
                    outp_h.at[cid, pl.ds(row0, RPT)])


@functools.lru_cache(maxsize=None)
def _e1_call():
    return functools.partial(
        pl.kernel,
        out_type=(jax.ShapeDtypeStruct((NC, NP, HID), _f32),
                  jax.ShapeDtypeStruct((NW, NP), _f32)),
        mesh=_sc_mesh(),
        compiler_params=pltpu.CompilerParams(needs_layout_passes=False),
        scratch_types=(
            pltpu.VMEM((H, HID), _f32),
            pltpu.VMEM((H, HID), _f32),
            pltpu.VMEM((H, HID), _f32),
            pltpu.VMEM((H, HID), _f32),
            pltpu.VMEM((H, HID), _f32),
            pltpu.VMEM((H, HID), _f32),
            pltpu.VMEM((2, SUP), _i32),
            pltpu.VMEM((2, SUP), _i32),
            pltpu.VMEM((HID,), _f32),
            pltpu.VMEM((NP,), _f32),
            pltpu.VMEM((H * 16,), _f32),
            pltpu.VMEM((16, HID), _f32),
            pltpu.VMEM_SHARED((NP, HID), _f32),
            pltpu.SemaphoreType.DMA,
            pltpu.SemaphoreType.DMA,
        ),
    )(_e1_body)


BA = 1280
BE = 2048


def _dot(a, b):
    return jnp.dot(a, b, preferred_element_type=_f32)


def _gat_head(x, asum, cnt, Wl, bl, Wr, br, We, att, xl_ref, xr_ref, s_ref):
    xl = _dot(x, Wl) + bl
    xr = _dot(x, Wr) + br
    c = jnp.maximum(jnp.sum(cnt, axis=1, keepdims=True), 1.0)
    la = (asum[0, :, :DE] + asum[1, :, :DE]) / c
    lv = xl + xr + _dot(la, We)
    lv = jnp.maximum(lv, 0.2 * lv)
    s_ref[...] = jnp.sum(lv * att, axis=1, keepdims=True)
    xl_ref[...] = xl
    xr_ref[...] = xr


def _a0_kernel(x_ref, asum_ref, cnt_ref, Wl_ref, bl_ref, Wr_ref, br_ref,
               We_ref, att_ref, xl_ref, xr_ref, s_ref):
    _gat_head(x_ref[...], asum_ref[...], cnt_ref[...], Wl_ref[...],
              bl_ref[...], Wr_ref[...], br_ref[...], We_ref[...],
              att_ref[...], xl_ref, xr_ref, s_ref)


def _finish(outp, denp, xlp, sprev, bprev):
    es = jnp.exp(sprev)
    den = jnp.sum(denp, axis=1, keepdims=True) + es
    return jnp.tanh((outp[0] + outp[1] + es * xlp) / den + bprev)


def _al_kernel(outp_ref, denp_ref, xlp_ref, sprev_ref, bprev_ref,
               asum_ref, cnt_ref, Wl_ref, bl_ref, Wr_ref, br_ref, We_ref,
               att_ref, xl_ref, xr_ref, s_ref):
    h = _finish(outp_ref[...], denp_ref[...], xlp_ref[...], sprev_ref[...],
                bprev_ref[...])
    _gat_head(h, asum_ref[...], cnt_ref[...], Wl_ref[...], bl_ref[...],
              Wr_ref[...], br_ref[...], We_ref[...], att_ref[...],
              xl_ref, xr_ref, s_ref)


def _a5_kernel(outp_ref, denp_ref, xlp_ref, sprev_ref, bprev_ref,
               Wg1_ref, bg1_ref, Wg2_ref, bg2_ref, wg3_ref, bg3_ref,
               Wa1_ref, ba1_ref, Wa2_ref, ba2_ref, g_ref, t_ref):
    h = _finish(outp_ref[...], denp_ref[...], xlp_ref[...], sprev_ref[...],
                bprev_ref[...])
    g1 = jnp.tanh(_dot(h, Wg1_ref[...]) + bg1_ref[...])
    g2 = jnp.tanh(_dot(g1, Wg2_ref[...]) + bg2_ref[...])
    g_ref[...] = (jnp.sum(g2 * wg3_ref[...], axis=1, keepdims=True)
                  + bg3_ref[...])
    t1 = jnp.tanh(_dot(h, Wa1_ref[...]) + ba1_ref[...])
    t_ref[...] = jnp.tanh(_dot(t1, Wa2_ref[...]) + ba2_ref[...])


def _ec_kernel(ea_ref, We_ref, out_ref):
    out_ref[...] = _dot(ea_ref[...], We_ref[...])


def _pool_kernel(g_ref, t_ref, bidr_ref, bidc_ref, W1_ref, b1_ref, W2_ref,
                 b2_ref, W3_ref, b3_ref, out_ref):
    gr = g_ref[...]
    bidr = bidr_ref[...]
    gid = lax.broadcasted_iota(_i32, (G, NP), 0)
    oh = bidr == gid
    ohf = oh.astype(_f32)
    m = jnp.max(jnp.where(oh, gr, -1e30), axis=1, keepdims=True)
    mb = jnp.sum(ohf * m, axis=0, keepdims=True)
    ex = jnp.where(bidr < G, jnp.exp(gr - mb), 0.0)
    den = jnp.sum(ohf * ex, axis=1, keepdims=True)
    t = jnp.where(bidc_ref[...] < G, t_ref[...], 0.0)
    num = _dot(ohf * ex, t)
    pooled = num / jnp.maximum(den, 1e-30)
    o = jnp.tanh(_dot(pooled, W1_ref[...]) + b1_ref[...])
    o = jnp.tanh(_dot(o, W2_ref[...]) + b2_ref[...])
    o = jnp.tanh(_dot(o, W3_ref[...]) + b3_ref[...])
    out_ref[...] = o


def _full(shape):
    ndim = len(shape)
    return pl.BlockSpec(shape, lambda *_, _n=ndim: (0,) * _n)


def _a_call(kern, extra_specs, *args):
    grid = NP // BA
    out_shapes = [jax.ShapeDtypeStruct((NP, HID), _f32),
                  jax.ShapeDtypeStruct((NP, HID), _f32),
                  jax.ShapeDtypeStruct((NP, 1), _f32)]
    out_specs = [pl.BlockSpec((BA, HID), lambda i: (i, 0)),
                 pl.BlockSpec((BA, HID), lambda i: (i, 0)),
                 pl.BlockSpec((BA, 1), lambda i: (i, 0))]
    return pl.pallas_call(
        kern, grid=(grid,), in_specs=list(extra_specs),
        out_specs=out_specs, out_shape=out_shapes,
    )(*args)


_SPEC_X = pl.BlockSpec((BA, HID), lambda i: (i, 0))
_SPEC_S = pl.BlockSpec((BA, 1), lambda i: (i, 0))
_SPEC_ASUM = pl.BlockSpec((NC, BA, HID), lambda i: (0, i, 0))
_SPEC_CNT = pl.BlockSpec((BA, NW), lambda i: (i, 0))
_SPEC_DEN = pl.BlockSpec((BA, NW), lambda i: (i, 0))
_SPEC_OUTP = pl.BlockSpec((NC, BA, HID), lambda i: (0, i, 0))
_SPEC_W = _full((HID, HID))
_SPEC_B = _full((HID,))
_SPEC_WE = _full((DE, HID))


def kernel(x, edge_index, edge_attr, batch_ids, params):
    src = edge_index[0]
    dst = edge_index[1]
    srcp = jnp.pad(src, (0, EP - E), constant_values=NP - 1)
    dstp = jnp.pad(dst, (0, EP - E), constant_values=NP - 1)
    eap = jnp.pad(edge_attr, ((0, EP - E), (0, 0)))
    xp = jnp.pad(x, ((0, NP - N), (0, 0)))
    bidp = jnp.pad(batch_ids, (0, NP - N), constant_values=G)

    cnt_p, asum_p = _p0_call()(src, dst, jnp.reshape(edge_attr, (E * DE,)))
    cnt_p = jnp.transpose(cnt_p)

    gps = params["gat"]
    outp = None
    denp = None
    xl_prev = None
    s_prev = None
    for l, p in enumerate(gps):
        if l == 0:
            xl, xr, s = _a_call(
                _a0_kernel,
                [_SPEC_X, _SPEC_ASUM, _SPEC_CNT, _SPEC_W, _SPEC_B, _SPEC_W,
                 _SPEC_B, _SPEC_WE, _SPEC_B],
                xp, asum_p, cnt_p, p["Wl"], p["bl"], p["Wr"], p["br"],
                p["We"], p["att"])
        else:
            bprev = gps[l - 1]["bias"]
            xl, xr, s = _a_call(
                _al_kernel,
                [_SPEC_OUTP, _SPEC_DEN, _SPEC_X, _SPEC_S, _SPEC_B,
                 _SPEC_ASUM, _SPEC_CNT, _SPEC_W, _SPEC_B, _SPEC_W, _SPEC_B,
                 _SPEC_WE, _SPEC_B],
                outp, denp, xl_prev, s_prev, bprev, asum_p, cnt_p,
                p["Wl"], p["bl"], p["Wr"], p["br"], p["We"], p["att"])

        ec = pl.pallas_call(
            _ec_kernel, grid=(EP // BE,),
            in_specs=[pl.BlockSpec((BE, DE), lambda i: (i, 0)), _SPEC_WE],
            out_specs=pl.BlockSpec((BE, HID), lambda i: (i, 0)),
            out_shape=jax.ShapeDtypeStruct((EP, HID), _f32),
        )(eap, p["We"])

        outp, denp = _e1_call()(xl, xr, ec, srcp, dstp, p["att"])
        denp = jnp.transpose(denp)
        xl_prev = xl
        s_prev = s

    (Wg1, bg1), (Wg2, bg2), (Wg3, bg3) = params["gate_nn"]
    (Wa1, ba1), (Wa2, ba2) = params["att_nn"]
    g, t = pl.pallas_call(
        _a5_kernel, grid=(NP // BA,),
        in_specs=[_SPEC_OUTP, _SPEC_DEN, _SPEC_X, _SPEC_S, _SPEC_B,
                  _SPEC_W, _SPEC_B, _SPEC_W, _SPEC_B, _SPEC_B, _full((1,)),
                  _SPEC_W, _SPEC_B, _SPEC_W, _SPEC_B],
        out_specs=[pl.BlockSpec((BA, 1), lambda i: (i, 0)), _SPEC_X],
        out_shape=[jax.ShapeDtypeStruct((NP, 1), _f32),
                   jax.ShapeDtypeStruct((NP, HID), _f32)],
    )(outp, denp, xl_prev, s_prev, gps[4]["bias"], Wg1, bg1, Wg2, bg2,
      Wg3[:, 0], bg3, Wa1, ba1, Wa2, ba2)

    (W1, b1), (W2, b2), (W3, b3) = params["fc"]
    out = pl.pallas_call(
        _pool_kernel,
        in_specs=[_full((1, NP)), _full((NP, HID)), _full((1, NP)),
                  _full((NP, 1)), _SPEC_W, _SPEC_B, _SPEC_W, _SPEC_B,
                  _full((HID, 1)), _full((1,))],
        out_specs=_full((G, 1)),
        out_shape=jax.ShapeDtypeStruct((G, 1), _f32),
    )(jnp.reshape(g, (1, NP)), t, jnp.reshape(bidp, (1, NP)),
      jnp.reshape(bidp, (NP, 1)), W1, b1, W2, b2, W3, b3)
    return out
